# baseline (speedup 1.0000x reference)
import functools

import jax
import jax.numpy as jnp
from jax.experimental import pallas as pl
from jax.experimental.pallas import tpu as pltpu

_VOCAB = 100000
_EMBED = 64
_CTX = 3
_HIDDEN = 128
_N = _CTX * _VOCAB
_BLK = 16384
_NBLK = (_N + _BLK - 1) // _BLK


def _fused_kernel(idx_ref, emb_ref, w1_ref, b1_ref, w2a_ref, w2b_ref,
                  w2c_ref, w2d_ref, b2_ref,
                  out_ref, acc_ref, h_ref, m_ref, s_ref):
    j = pl.program_id(0)

    @pl.when(j == 0)
    def _init():
        row = emb_ref[pl.ds(idx_ref[0] % 8, 1), :]
        h = jnp.dot(row, w1_ref[...],
                    preferred_element_type=jnp.float32) + b1_ref[...]
        h_ref[...] = jnp.transpose(jnp.maximum(h, 0.0))
        m_ref[0] = -jnp.inf
        s_ref[0] = 0.0

    acc8 = None
    for t, ref in enumerate((w2a_ref, w2b_ref, w2c_ref, w2d_ref)):
        for i in range(4):
            r0 = 8 * i
            prod = ref[r0:r0 + 8, :] * h_ref[32 * t + r0:32 * t + r0 + 8, :]
            acc8 = prod if acc8 is None else acc8 + prod
    chunk = jnp.sum(acc8, axis=0, keepdims=True)
    chunk = chunk + b2_ref[...][None, :]
    col = j * _BLK + jax.lax.broadcasted_iota(jnp.int32, (1, _BLK), 1)
    chunk = jnp.where(col < _N, chunk, -jnp.inf)
    acc_ref[:, pl.ds(j * _BLK, _BLK)] = chunk

    m_old = m_ref[0]
    m_new = jnp.maximum(m_old, jnp.max(chunk))
    s_ref[0] = s_ref[0] * jnp.exp(m_old - m_new) + jnp.sum(
        jnp.exp(chunk - m_new))
    m_ref[0] = m_new

    @pl.when(j == _NBLK - 1)
    def _finish():
        logz = m_ref[0] + jnp.log(s_ref[0])
        out_ref[0:1, :] = acc_ref[:, 0:_VOCAB] - logz
        out_ref[1:2, :] = acc_ref[:, _VOCAB:2 * _VOCAB] - logz
        out_ref[2:3, :] = acc_ref[:, 2 * _VOCAB:3 * _VOCAB] - logz


def kernel(inputs, emb_table, W1, b1, W2, b2):
    idx = inputs.astype(jnp.int32)
    return pl.pallas_call(
        _fused_kernel,
        grid_spec=pltpu.PrefetchScalarGridSpec(
            num_scalar_prefetch=1,
            grid=(_NBLK,),
            in_specs=[
                pl.BlockSpec((8, _EMBED), lambda j, idx: (idx[0] // 8, 0)),
                pl.BlockSpec((_EMBED, _HIDDEN), lambda j, idx: (0, 0)),
                pl.BlockSpec((1, _HIDDEN), lambda j, idx: (0, 0)),
                pl.BlockSpec((32, _BLK), lambda j, idx: (0, j)),
                pl.BlockSpec((32, _BLK), lambda j, idx: (1, j)),
                pl.BlockSpec((32, _BLK), lambda j, idx: (2, j)),
                pl.BlockSpec((32, _BLK), lambda j, idx: (3, j)),
                pl.BlockSpec((_BLK,), lambda j, idx: (j,)),
            ],
            out_specs=pl.BlockSpec((_CTX, _VOCAB), lambda j, idx: (0, 0)),
            scratch_shapes=[
                pltpu.VMEM((1, _NBLK * _BLK), jnp.float32),
                pltpu.VMEM((_HIDDEN, 1), jnp.float32),
                pltpu.SMEM((1,), jnp.float32),
                pltpu.SMEM((1,), jnp.float32),
            ],
        ),
        out_shape=jax.ShapeDtypeStruct((_CTX, _VOCAB), jnp.float32),
        compiler_params=pltpu.CompilerParams(
            dimension_semantics=("arbitrary",),
        ),
    )(idx, emb_table, W1, b1.reshape(1, _HIDDEN), W2, W2, W2, W2, b2)


# speedup vs baseline: 2.1268x; 2.1268x over previous
import jax
import jax.numpy as jnp
from jax.experimental import pallas as pl
from jax.experimental.pallas import tpu as pltpu

_VOCAB = 100000
_EMBED = 64
_CTX = 3
_HIDDEN = 128
_N = _CTX * _VOCAB
_BLK = 4096
_NBLK = (_N + _BLK - 1) // _BLK


def _fused_kernel(idx_ref, embt_ref, w1_ref, b1_ref, w2t_ref, b2_ref,
                  out_ref, acc_ref, h_ref, m_ref, s_ref):
    j = pl.program_id(0)

    @pl.when(j == 0)
    def _init():
        lane = jax.lax.broadcasted_iota(jnp.int32, (_EMBED, 128), 1)
        sel = jnp.where(lane == idx_ref[0] % 128, embt_ref[...], 0.0)
        emb_row = jnp.sum(sel, axis=1, keepdims=True).T
        h = jnp.dot(emb_row, w1_ref[...],
                    preferred_element_type=jnp.float32) + b1_ref[...]
        h_ref[...] = jnp.maximum(h, 0.0).T
        m_ref[0] = -jnp.inf
        s_ref[0] = 0.0

    chunk_col = jnp.dot(w2t_ref[...], h_ref[...],
                        preferred_element_type=jnp.float32)
    chunk = chunk_col.T + b2_ref[...][None, :]
    col = j * _BLK + jax.lax.broadcasted_iota(jnp.int32, (1, _BLK), 1)
    chunk = jnp.where(col < _N, chunk, -jnp.inf)
    acc_ref[:, pl.ds(j * _BLK, _BLK)] = chunk

    m_old = m_ref[0]
    m_new = jnp.maximum(m_old, jnp.max(chunk))
    s_ref[0] = s_ref[0] * jnp.exp(m_old - m_new) + jnp.sum(
        jnp.exp(chunk - m_new))
    m_ref[0] = m_new

    @pl.when(j == _NBLK - 1)
    def _finish():
        logz = m_ref[0] + jnp.log(s_ref[0])
        out_ref[0:1, :] = acc_ref[:, 0:_VOCAB] - logz
        out_ref[1:2, :] = acc_ref[:, _VOCAB:2 * _VOCAB] - logz
        out_ref[2:3, :] = acc_ref[:, 2 * _VOCAB:3 * _VOCAB] - logz


def kernel(inputs, emb_table, W1, b1, W2, b2):
    idx = inputs.astype(jnp.int32)
    return pl.pallas_call(
        _fused_kernel,
        grid_spec=pltpu.PrefetchScalarGridSpec(
            num_scalar_prefetch=1,
            grid=(_NBLK,),
            in_specs=[
                pl.BlockSpec((_EMBED, 128), lambda j, idx: (0, idx[0] // 128)),
                pl.BlockSpec((_EMBED, _HIDDEN), lambda j, idx: (0, 0)),
                pl.BlockSpec((1, _HIDDEN), lambda j, idx: (0, 0)),
                pl.BlockSpec((_BLK, _HIDDEN), lambda j, idx: (j, 0)),
                pl.BlockSpec((_BLK,), lambda j, idx: (j,)),
            ],
            out_specs=pl.BlockSpec((_CTX, _VOCAB), lambda j, idx: (0, 0)),
            scratch_shapes=[
                pltpu.VMEM((1, _NBLK * _BLK), jnp.float32),
                pltpu.VMEM((_HIDDEN, 1), jnp.float32),
                pltpu.SMEM((1,), jnp.float32),
                pltpu.SMEM((1,), jnp.float32),
            ],
        ),
        out_shape=jax.ShapeDtypeStruct((_CTX, _VOCAB), jnp.float32),
        compiler_params=pltpu.CompilerParams(
            dimension_semantics=("arbitrary",),
        ),
    )(idx, emb_table.T, W1, b1.reshape(1, _HIDDEN), W2.T, b2)


# speedup vs baseline: 3.3885x; 1.5932x over previous
import jax
import jax.numpy as jnp
from jax.experimental import pallas as pl
from jax.experimental.pallas import tpu as pltpu

_VOCAB = 100000
_EMBED = 64
_CTX = 3
_HIDDEN = 128
_N = _CTX * _VOCAB
_BLK = 16384
_NBLK = (_N + _BLK - 1) // _BLK


def _fused_kernel(idx_ref, embt_ref, w1_ref, b1_ref, w2t_ref, b2_ref,
                  out_ref, acc_ref, h_ref, m_ref, s_ref):
    j = pl.program_id(0)

    @pl.when(j == 0)
    def _init():
        lane = jax.lax.broadcasted_iota(jnp.int32, (_EMBED, 128), 1)
        sel = jnp.where(lane == idx_ref[0] % 128, embt_ref[...], 0.0)
        emb_row = jnp.sum(sel, axis=1, keepdims=True).T
        h = jnp.dot(emb_row, w1_ref[...],
                    preferred_element_type=jnp.float32) + b1_ref[...]
        h_ref[...] = jnp.maximum(h, 0.0).T
        m_ref[0] = -jnp.inf
        s_ref[0] = 0.0

    chunk_col = jnp.dot(w2t_ref[...], h_ref[...],
                        preferred_element_type=jnp.float32)
    chunk = chunk_col.T + b2_ref[...][None, :]
    col = j * _BLK + jax.lax.broadcasted_iota(jnp.int32, (1, _BLK), 1)
    chunk = jnp.where(col < _N, chunk, -jnp.inf)
    acc_ref[:, pl.ds(j * _BLK, _BLK)] = chunk

    m_old = m_ref[0]
    m_new = jnp.maximum(m_old, jnp.max(chunk))
    s_ref[0] = s_ref[0] * jnp.exp(m_old - m_new) + jnp.sum(
        jnp.exp(chunk - m_new))
    m_ref[0] = m_new

    @pl.when(j == _NBLK - 1)
    def _finish():
        logz = m_ref[0] + jnp.log(s_ref[0])
        out_ref[0:1, :] = acc_ref[:, 0:_VOCAB] - logz
        out_ref[1:2, :] = acc_ref[:, _VOCAB:2 * _VOCAB] - logz
        out_ref[2:3, :] = acc_ref[:, 2 * _VOCAB:3 * _VOCAB] - logz


def kernel(inputs, emb_table, W1, b1, W2, b2):
    idx = inputs.astype(jnp.int32)
    return pl.pallas_call(
        _fused_kernel,
        grid_spec=pltpu.PrefetchScalarGridSpec(
            num_scalar_prefetch=1,
            grid=(_NBLK,),
            in_specs=[
                pl.BlockSpec((_EMBED, 128), lambda j, idx: (0, idx[0] // 128)),
                pl.BlockSpec((_EMBED, _HIDDEN), lambda j, idx: (0, 0)),
                pl.BlockSpec((1, _HIDDEN), lambda j, idx: (0, 0)),
                pl.BlockSpec((_BLK, _HIDDEN), lambda j, idx: (j, 0)),
                pl.BlockSpec((_BLK,), lambda j, idx: (j,)),
            ],
            out_specs=pl.BlockSpec((_CTX, _VOCAB), lambda j, idx: (0, 0)),
            scratch_shapes=[
                pltpu.VMEM((1, _NBLK * _BLK), jnp.float32),
                pltpu.VMEM((_HIDDEN, 1), jnp.float32),
                pltpu.SMEM((1,), jnp.float32),
                pltpu.SMEM((1,), jnp.float32),
            ],
        ),
        out_shape=jax.ShapeDtypeStruct((_CTX, _VOCAB), jnp.float32),
        compiler_params=pltpu.CompilerParams(
            dimension_semantics=("arbitrary",),
        ),
    )(idx, emb_table.T, W1, b1.reshape(1, _HIDDEN), W2.T, b2)


# speedup vs baseline: 3.5339x; 1.0429x over previous
import jax
import jax.numpy as jnp
from jax.experimental import pallas as pl
from jax.experimental.pallas import tpu as pltpu

_VOCAB = 100000
_EMBED = 64
_CTX = 3
_HIDDEN = 128
_N = _CTX * _VOCAB
_BLK = 16384
_NBLK = (_N + _BLK - 1) // _BLK


def _fused_kernel(idx_ref, embt_ref, w1_ref, b1_ref, w2t_ref, b2_ref,
                  out_ref, acc_ref, h_ref, m_ref, s_ref):
    j = pl.program_id(0)

    @pl.when(j == 0)
    def _init():
        lane = jax.lax.broadcasted_iota(jnp.int32, (_EMBED, 128), 1)
        sel = jnp.where(lane == idx_ref[0] % 128, embt_ref[...], 0.0)
        emb_row = jnp.sum(sel, axis=1, keepdims=True).T
        h = jnp.dot(emb_row, w1_ref[...],
                    preferred_element_type=jnp.float32) + b1_ref[...]
        h_ref[...] = jnp.maximum(h, 0.0).T
        m_ref[0] = -jnp.inf
        s_ref[0] = 0.0

    chunk_col = jnp.dot(w2t_ref[...], h_ref[...],
                        preferred_element_type=jnp.float32)
    chunk = chunk_col.T + b2_ref[...][None, :]
    acc_ref[:, pl.ds(j * _BLK, _BLK)] = chunk

    def _update_stats(c):
        m_old = m_ref[0]
        m_new = jnp.maximum(m_old, jnp.max(c))
        s_ref[0] = s_ref[0] * jnp.exp(m_old - m_new) + jnp.sum(
            jnp.exp(c - m_new))
        m_ref[0] = m_new

    @pl.when(j < _NBLK - 1)
    def _stats_full():
        _update_stats(chunk)

    @pl.when(j == _NBLK - 1)
    def _stats_masked():
        col = jax.lax.broadcasted_iota(jnp.int32, (1, _BLK), 1)
        _update_stats(jnp.where(col < _N - (_NBLK - 1) * _BLK,
                                chunk, -jnp.inf))

    @pl.when(j == _NBLK - 1)
    def _finish():
        logz = m_ref[0] + jnp.log(s_ref[0])
        out_ref[0:1, :] = acc_ref[:, 0:_VOCAB] - logz
        out_ref[1:2, :] = acc_ref[:, _VOCAB:2 * _VOCAB] - logz
        out_ref[2:3, :] = acc_ref[:, 2 * _VOCAB:3 * _VOCAB] - logz


def kernel(inputs, emb_table, W1, b1, W2, b2):
    idx = inputs.astype(jnp.int32)
    return pl.pallas_call(
        _fused_kernel,
        grid_spec=pltpu.PrefetchScalarGridSpec(
            num_scalar_prefetch=1,
            grid=(_NBLK,),
            in_specs=[
                pl.BlockSpec((_EMBED, 128), lambda j, idx: (0, idx[0] // 128)),
                pl.BlockSpec((_EMBED, _HIDDEN), lambda j, idx: (0, 0)),
                pl.BlockSpec((1, _HIDDEN), lambda j, idx: (0, 0)),
                pl.BlockSpec((_BLK, _HIDDEN), lambda j, idx: (j, 0)),
                pl.BlockSpec((_BLK,), lambda j, idx: (j,)),
            ],
            out_specs=pl.BlockSpec((_CTX, _VOCAB), lambda j, idx: (0, 0)),
            scratch_shapes=[
                pltpu.VMEM((1, _NBLK * _BLK), jnp.float32),
                pltpu.VMEM((_HIDDEN, 1), jnp.float32),
                pltpu.SMEM((1,), jnp.float32),
                pltpu.SMEM((1,), jnp.float32),
            ],
        ),
        out_shape=jax.ShapeDtypeStruct((_CTX, _VOCAB), jnp.float32),
        compiler_params=pltpu.CompilerParams(
            dimension_semantics=("arbitrary",),
        ),
    )(idx, emb_table.T, W1, b1.reshape(1, _HIDDEN), W2.T, b2)
